# baseline (speedup 1.0000x reference)
import jax
import jax.numpy as jnp
from jax.experimental import pallas as pl
from jax.experimental.pallas import tpu as pltpu

N = 4096
FEAT = 128
HID = 16
NH = 9
NC = 27
BLK = 512


def _body(x_ref, adj_ref, w_gc_ref, b_gc_ref, w_mlp_ref, b_mlp_ref,
          w_cls_ref, b_cls_ref, out_ref, sup_ref, acc_ref):
    i = pl.program_id(0)
    b = pl.program_id(1)

    @pl.when(b == 0)
    def _():
        sup_ref[...] = jnp.dot(x_ref[i], w_gc_ref[i],
                             preferred_element_type=jnp.float32)

    adj = adj_ref[0]
    adj = jnp.where(jnp.isnan(adj), 0.0, adj)
    h = jnp.dot(adj, sup_ref[...], preferred_element_type=jnp.float32)
    h = h + b_gc_ref[i]
    t = jnp.tanh(jnp.dot(h, w_mlp_ref[i], preferred_element_type=jnp.float32)
                 + b_mlp_ref[i])
    w_cls_i = w_cls_ref[pl.ds(i * NH, NH), :]
    contrib = jnp.dot(t, w_cls_i, preferred_element_type=jnp.float32)

    @pl.when(i == 0)
    def _():
        acc_ref[pl.ds(b * BLK, BLK), :] = contrib + b_cls_ref[0]

    @pl.when(i == 1)
    def _():
        acc_ref[pl.ds(b * BLK, BLK), :] += contrib

    @pl.when(i == 2)
    def _():
        out_ref[...] = acc_ref[pl.ds(b * BLK, BLK), :] + contrib


@jax.jit
def kernel(x, adjs, W_gc, b_gc, W_mlp, b_mlp, W_cls, b_cls):
    nb = N // BLK
    out = pl.pallas_call(
        _body,
        grid=(3, nb),
        in_specs=[
            pl.BlockSpec((3, N, FEAT), lambda i, b: (0, 0, 0)),
            pl.BlockSpec((1, BLK, N), lambda i, b: (i, b, 0)),
            pl.BlockSpec((3, FEAT, HID), lambda i, b: (0, 0, 0)),
            pl.BlockSpec((3, HID), lambda i, b: (0, 0)),
            pl.BlockSpec((3, HID, NH), lambda i, b: (0, 0, 0)),
            pl.BlockSpec((3, NH), lambda i, b: (0, 0)),
            pl.BlockSpec((3 * NH, NC), lambda i, b: (0, 0)),
            pl.BlockSpec((1, NC), lambda i, b: (0, 0)),
        ],
        out_specs=pl.BlockSpec((BLK, NC), lambda i, b: (b, 0)),
        out_shape=jax.ShapeDtypeStruct((N, NC), jnp.float32),
        scratch_shapes=[pltpu.VMEM((N, HID), jnp.float32),
                        pltpu.VMEM((N, NC), jnp.float32)],
    )(x, adjs, W_gc, b_gc, W_mlp, b_mlp, W_cls, b_cls.reshape(1, NC))
    return out


# speedup vs baseline: 1.0174x; 1.0174x over previous
import jax
import jax.numpy as jnp
from jax.experimental import pallas as pl
from jax.experimental.pallas import tpu as pltpu

N = 4096
FEAT = 128
HID = 16
NH = 9
NC = 27
BLK = 512


def _body(x_ref, adj_ref, w_gc_ref, b_gc_ref, w_mlp_ref, b_mlp_ref,
          w_cls_ref, b_cls_ref, out_ref, sup_ref):
    b = pl.program_id(0)
    i = pl.program_id(1)

    @pl.when(b == 0)
    def _():
        sup_ref[i] = jnp.dot(x_ref[i], w_gc_ref[i],
                             preferred_element_type=jnp.float32)

    adj = adj_ref[0]
    adj = jnp.where(jnp.isnan(adj), 0.0, adj)
    h = jnp.dot(adj, sup_ref[i], preferred_element_type=jnp.float32)
    h = h + b_gc_ref[i]
    t = jnp.tanh(jnp.dot(h, w_mlp_ref[i], preferred_element_type=jnp.float32)
                 + b_mlp_ref[i])
    w_cls_i = w_cls_ref[pl.ds(i * NH, NH), :]
    contrib = jnp.dot(t, w_cls_i, preferred_element_type=jnp.float32)

    @pl.when(i == 0)
    def _():
        out_ref[...] = contrib + b_cls_ref[0]

    @pl.when(i != 0)
    def _():
        out_ref[...] += contrib


@jax.jit
def kernel(x, adjs, W_gc, b_gc, W_mlp, b_mlp, W_cls, b_cls):
    nb = N // BLK
    out = pl.pallas_call(
        _body,
        grid=(nb, 3),
        in_specs=[
            pl.BlockSpec((3, N, FEAT), lambda b, i: (0, 0, 0)),
            pl.BlockSpec((1, BLK, N), lambda b, i: (i, b, 0)),
            pl.BlockSpec((3, FEAT, HID), lambda b, i: (0, 0, 0)),
            pl.BlockSpec((3, HID), lambda b, i: (0, 0)),
            pl.BlockSpec((3, HID, NH), lambda b, i: (0, 0, 0)),
            pl.BlockSpec((3, NH), lambda b, i: (0, 0)),
            pl.BlockSpec((3 * NH, NC), lambda b, i: (0, 0)),
            pl.BlockSpec((1, NC), lambda b, i: (0, 0)),
        ],
        out_specs=pl.BlockSpec((BLK, NC), lambda b, i: (b, 0)),
        out_shape=jax.ShapeDtypeStruct((N, NC), jnp.float32),
        scratch_shapes=[pltpu.VMEM((3, N, HID), jnp.float32)],
    )(x, adjs, W_gc, b_gc, W_mlp, b_mlp, W_cls, b_cls.reshape(1, NC))
    return out


# speedup vs baseline: 1.0235x; 1.0060x over previous
import jax
import jax.numpy as jnp
from jax.experimental import pallas as pl
from jax.experimental.pallas import tpu as pltpu

N = 4096
FEAT = 128
HID = 16
NH = 9
NC = 27
BLK = 1024


def _body(x_ref, adj_ref, w_gc_ref, b_gc_ref, w_mlp_ref, b_mlp_ref,
          w_cls_ref, b_cls_ref, out_ref, sup_ref):
    b = pl.program_id(0)
    i = pl.program_id(1)

    @pl.when(b == 0)
    def _():
        sup_ref[i] = jnp.dot(x_ref[i], w_gc_ref[i],
                             preferred_element_type=jnp.float32)

    adj = adj_ref[0]
    adj = jnp.where(jnp.isnan(adj), 0.0, adj)
    h = jnp.dot(adj, sup_ref[i], preferred_element_type=jnp.float32)
    h = h + b_gc_ref[i]
    t = jnp.tanh(jnp.dot(h, w_mlp_ref[i], preferred_element_type=jnp.float32)
                 + b_mlp_ref[i])
    w_cls_i = w_cls_ref[pl.ds(i * NH, NH), :]
    contrib = jnp.dot(t, w_cls_i, preferred_element_type=jnp.float32)

    @pl.when(i == 0)
    def _():
        out_ref[...] = contrib + b_cls_ref[0]

    @pl.when(i != 0)
    def _():
        out_ref[...] += contrib


@jax.jit
def kernel(x, adjs, W_gc, b_gc, W_mlp, b_mlp, W_cls, b_cls):
    nb = N // BLK
    out = pl.pallas_call(
        _body,
        grid=(nb, 3),
        in_specs=[
            pl.BlockSpec((3, N, FEAT), lambda b, i: (0, 0, 0)),
            pl.BlockSpec((1, BLK, N), lambda b, i: (i, b, 0)),
            pl.BlockSpec((3, FEAT, HID), lambda b, i: (0, 0, 0)),
            pl.BlockSpec((3, HID), lambda b, i: (0, 0)),
            pl.BlockSpec((3, HID, NH), lambda b, i: (0, 0, 0)),
            pl.BlockSpec((3, NH), lambda b, i: (0, 0)),
            pl.BlockSpec((3 * NH, NC), lambda b, i: (0, 0)),
            pl.BlockSpec((1, NC), lambda b, i: (0, 0)),
        ],
        out_specs=pl.BlockSpec((BLK, NC), lambda b, i: (b, 0)),
        out_shape=jax.ShapeDtypeStruct((N, NC), jnp.float32),
        scratch_shapes=[pltpu.VMEM((3, N, HID), jnp.float32)],
    )(x, adjs, W_gc, b_gc, W_mlp, b_mlp, W_cls, b_cls.reshape(1, NC))
    return out
